# baseline (device time: 27129 ns/iter reference)
import jax
import jax.numpy as jnp
from jax import lax
from jax.experimental import pallas as pl
from jax.experimental.pallas import tpu as pltpu

N_DEV = 8
SQ = 256
D = 1024
DH = 128
HQ_PER = 8
KV_COLS = 256
CH = SQ // N_DEV
SCALE = 0.08838834764831843


def kernel(x, Wq, Wo, Wk, Wv):

    def body(x_ref, wq_ref, wo_ref, wk_ref, wv_ref, out_ref,
             wkv_v, wo_v, pbuf32, pbuf, scatter_buf, bbuf, bcast_buf,
             load_sems, ssend_sems, srecv_sems, bsend_sems, brecv_sems):
        my = lax.axis_index("i")
        acc = out_ref.at[0]
        my_rows = pl.ds(CH * my, CH)

        kcopy = pltpu.make_async_copy(
            wk_ref.at[:, pl.ds(my * KV_COLS, KV_COLS)],
            wkv_v.at[0], load_sems.at[0])
        vcopy = pltpu.make_async_copy(
            wv_ref.at[:, pl.ds(my * KV_COLS, KV_COLS)],
            wkv_v.at[1], load_sems.at[1])
        wocopy = pltpu.make_async_copy(wo_ref, wo_v, load_sems.at[2])
        kcopy.start()
        vcopy.start()
        wocopy.start()

        barrier = pltpu.get_barrier_semaphore()
        for p in range(N_DEV):
            pl.when(my != p)(lambda p=p: pl.semaphore_signal(
                barrier, inc=1,
                device_id=(p,), device_id_type=pl.DeviceIdType.MESH,
            ))

        xv = x_ref[0, :, :].astype(jnp.bfloat16)
        q = jnp.dot(xv, wq_ref[...].astype(jnp.bfloat16),
                    preferred_element_type=jnp.float32)
        kcopy.wait()
        vcopy.wait()
        k = jnp.dot(xv, wkv_v[0].astype(jnp.bfloat16),
                    preferred_element_type=jnp.float32)
        v = jnp.dot(xv, wkv_v[1].astype(jnp.bfloat16),
                    preferred_element_type=jnp.float32)

        outs = []
        for h in range(HQ_PER):
            qh = q[:, h * DH:(h + 1) * DH].astype(jnp.bfloat16)
            g = h // 4
            kh = k[:, g * DH:(g + 1) * DH].astype(jnp.bfloat16)
            vh = v[:, g * DH:(g + 1) * DH].astype(jnp.bfloat16)
            s = lax.dot_general(
                qh, kh, (((1,), (1,)), ((), ())),
                preferred_element_type=jnp.float32,
            ) * SCALE
            m = jnp.max(s, axis=-1, keepdims=True)
            p = jnp.exp(s - m)
            l = jnp.sum(p, axis=-1, keepdims=True)
            ph = p.astype(jnp.bfloat16)
            outs.append(jnp.dot(ph, vh, preferred_element_type=jnp.float32) / l)
        o = jnp.concatenate(outs, axis=1).astype(jnp.bfloat16)
        wocopy.wait()
        wo_bf = wo_v[...].astype(jnp.bfloat16)

        pl.semaphore_wait(barrier, N_DEV - 1)

        for c in range(N_DEV):
            rows = slice(CH * c, CH * (c + 1))
            chunk = jnp.dot(o[rows, :], wo_bf,
                            preferred_element_type=jnp.float32)
            pbuf32[rows, :] = chunk
            pbuf[rows, :] = chunk.astype(jnp.bfloat16)

            def p1_send(c=c):
                rdma = pltpu.make_async_remote_copy(
                    src_ref=pbuf.at[pl.ds(CH * c, CH), :],
                    dst_ref=scatter_buf.at[my],
                    send_sem=ssend_sems.at[c],
                    recv_sem=srecv_sems.at[my],
                    device_id=(c,),
                    device_id_type=pl.DeviceIdType.MESH,
                )
                rdma.start()
            pl.when(my != c)(p1_send)

        scatter_buf[my] = jnp.zeros((CH, D), jnp.bfloat16)
        for j in range(N_DEV):
            def p1_wait(j=j):
                recv = pltpu.make_async_remote_copy(
                    src_ref=scatter_buf.at[j],
                    dst_ref=scatter_buf.at[j],
                    send_sem=ssend_sems.at[j],
                    recv_sem=srecv_sems.at[j],
                    device_id=(j,),
                    device_id_type=pl.DeviceIdType.MESH,
                )
                recv.wait_recv()
            pl.when(my != j)(p1_wait)

        red = pbuf32[my_rows, :]
        for j in range(N_DEV):
            red = red + scatter_buf[j].astype(jnp.float32)
        acc[my_rows, :] = red
        bbuf[...] = red.astype(jnp.bfloat16)

        for c in range(N_DEV):
            def p2_send(c=c):
                rdma = pltpu.make_async_remote_copy(
                    src_ref=bbuf,
                    dst_ref=bcast_buf.at[my],
                    send_sem=bsend_sems.at[c],
                    recv_sem=brecv_sems.at[my],
                    device_id=(c,),
                    device_id_type=pl.DeviceIdType.MESH,
                )
                rdma.start()
            pl.when(my != c)(p2_send)

        for j in range(N_DEV):
            def p2_take(j=j):
                recv = pltpu.make_async_remote_copy(
                    src_ref=bcast_buf.at[j],
                    dst_ref=bcast_buf.at[j],
                    send_sem=bsend_sems.at[j],
                    recv_sem=brecv_sems.at[j],
                    device_id=(j,),
                    device_id_type=pl.DeviceIdType.MESH,
                )
                recv.wait_recv()
                acc[CH * j:CH * (j + 1), :] = (
                    bcast_buf[j].astype(jnp.float32))
            pl.when(my != j)(p2_take)

        for c in range(N_DEV):
            def drain(c=c):
                s1 = pltpu.make_async_remote_copy(
                    src_ref=pbuf.at[pl.ds(CH * c, CH), :],
                    dst_ref=scatter_buf.at[my],
                    send_sem=ssend_sems.at[c],
                    recv_sem=srecv_sems.at[my],
                    device_id=(c,),
                    device_id_type=pl.DeviceIdType.MESH,
                )
                s1.wait_send()
                s2 = pltpu.make_async_remote_copy(
                    src_ref=bbuf,
                    dst_ref=bcast_buf.at[my],
                    send_sem=bsend_sems.at[c],
                    recv_sem=brecv_sems.at[my],
                    device_id=(c,),
                    device_id_type=pl.DeviceIdType.MESH,
                )
                s2.wait_send()
            pl.when(my != c)(drain)

    return pl.pallas_call(
        body,
        out_shape=jax.ShapeDtypeStruct((1, SQ, D), jnp.float32),
        in_specs=[
            pl.BlockSpec(memory_space=pltpu.VMEM),
            pl.BlockSpec(memory_space=pltpu.VMEM),
            pl.BlockSpec(memory_space=pl.ANY),
            pl.BlockSpec(memory_space=pl.ANY),
            pl.BlockSpec(memory_space=pl.ANY),
        ],
        out_specs=pl.BlockSpec(memory_space=pltpu.VMEM),
        scratch_shapes=[
            pltpu.VMEM((2, D, KV_COLS), jnp.float32),
            pltpu.VMEM((D, D), jnp.float32),
            pltpu.VMEM((SQ, D), jnp.float32),
            pltpu.VMEM((SQ, D), jnp.bfloat16),
            pltpu.VMEM((N_DEV, CH, D), jnp.bfloat16),
            pltpu.VMEM((CH, D), jnp.bfloat16),
            pltpu.VMEM((N_DEV, CH, D), jnp.bfloat16),
            pltpu.SemaphoreType.DMA((3,)),
            pltpu.SemaphoreType.DMA((N_DEV,)),
            pltpu.SemaphoreType.DMA((N_DEV,)),
            pltpu.SemaphoreType.DMA((N_DEV,)),
            pltpu.SemaphoreType.DMA((N_DEV,)),
        ],
        compiler_params=pltpu.CompilerParams(collective_id=0),
    )(x, Wq, Wo, Wk, Wv)


# device time: 27072 ns/iter; 1.0021x vs baseline; 1.0021x over previous
import jax
import jax.numpy as jnp
from jax import lax
from jax.experimental import pallas as pl
from jax.experimental.pallas import tpu as pltpu

N_DEV = 8
SQ = 256
D = 1024
DH = 128
HQ_PER = 8
KV_COLS = 256
CH = SQ // N_DEV
SCALE = 0.08838834764831843


def kernel(x, Wq, Wo, Wk, Wv):

    def body(x_ref, wq_ref, wo_ref, wk_ref, wv_ref, out_ref,
             wkv_v, pbuf32, pbuf, scatter_buf, bbuf, bcast_buf,
             load_sems, ssend_sems, srecv_sems, bsend_sems, brecv_sems):
        my = lax.axis_index("i")
        acc = out_ref.at[0]
        my_rows = pl.ds(CH * my, CH)

        kcopy = pltpu.make_async_copy(
            wk_ref.at[:, pl.ds(my * KV_COLS, KV_COLS)],
            wkv_v.at[0], load_sems.at[0])
        vcopy = pltpu.make_async_copy(
            wv_ref.at[:, pl.ds(my * KV_COLS, KV_COLS)],
            wkv_v.at[1], load_sems.at[1])
        kcopy.start()
        vcopy.start()

        barrier = pltpu.get_barrier_semaphore()
        for p in range(N_DEV):
            pl.when(my != p)(lambda p=p: pl.semaphore_signal(
                barrier, inc=1,
                device_id=(p,), device_id_type=pl.DeviceIdType.MESH,
            ))

        xv = x_ref[0, :, :].astype(jnp.bfloat16)
        q = jnp.dot(xv, wq_ref[...].astype(jnp.bfloat16),
                    preferred_element_type=jnp.float32)
        kcopy.wait()
        vcopy.wait()
        k = jnp.dot(xv, wkv_v[0].astype(jnp.bfloat16),
                    preferred_element_type=jnp.float32)
        v = jnp.dot(xv, wkv_v[1].astype(jnp.bfloat16),
                    preferred_element_type=jnp.float32)

        outs = []
        for h in range(HQ_PER):
            qh = q[:, h * DH:(h + 1) * DH].astype(jnp.bfloat16)
            g = h // 4
            kh = k[:, g * DH:(g + 1) * DH].astype(jnp.bfloat16)
            vh = v[:, g * DH:(g + 1) * DH].astype(jnp.bfloat16)
            s = lax.dot_general(
                qh, kh, (((1,), (1,)), ((), ())),
                preferred_element_type=jnp.float32,
            ) * SCALE
            m = jnp.max(s, axis=-1, keepdims=True)
            p = jnp.exp(s - m)
            l = jnp.sum(p, axis=-1, keepdims=True)
            ph = p.astype(jnp.bfloat16)
            outs.append(jnp.dot(ph, vh, preferred_element_type=jnp.float32) / l)
        o = jnp.concatenate(outs, axis=1).astype(jnp.bfloat16)
        wo_bf = wo_ref[...].astype(jnp.bfloat16)

        pl.semaphore_wait(barrier, N_DEV - 1)

        for c in range(N_DEV):
            rows = slice(CH * c, CH * (c + 1))
            chunk = jnp.dot(o[rows, :], wo_bf,
                            preferred_element_type=jnp.float32)
            pbuf32[rows, :] = chunk
            pbuf[rows, :] = chunk.astype(jnp.bfloat16)

            def p1_send(c=c):
                rdma = pltpu.make_async_remote_copy(
                    src_ref=pbuf.at[pl.ds(CH * c, CH), :],
                    dst_ref=scatter_buf.at[my],
                    send_sem=ssend_sems.at[c],
                    recv_sem=srecv_sems.at[my],
                    device_id=(c,),
                    device_id_type=pl.DeviceIdType.MESH,
                )
                rdma.start()
            pl.when(my != c)(p1_send)

        scatter_buf[my] = jnp.zeros((CH, D), jnp.bfloat16)
        for j in range(N_DEV):
            def p1_wait(j=j):
                recv = pltpu.make_async_remote_copy(
                    src_ref=scatter_buf.at[j],
                    dst_ref=scatter_buf.at[j],
                    send_sem=ssend_sems.at[j],
                    recv_sem=srecv_sems.at[j],
                    device_id=(j,),
                    device_id_type=pl.DeviceIdType.MESH,
                )
                recv.wait_recv()
            pl.when(my != j)(p1_wait)

        red = pbuf32[my_rows, :]
        for j in range(N_DEV):
            red = red + scatter_buf[j].astype(jnp.float32)
        acc[my_rows, :] = red
        bbuf[...] = red.astype(jnp.bfloat16)

        for c in range(N_DEV):
            def p2_send(c=c):
                rdma = pltpu.make_async_remote_copy(
                    src_ref=bbuf,
                    dst_ref=bcast_buf.at[my],
                    send_sem=bsend_sems.at[c],
                    recv_sem=brecv_sems.at[my],
                    device_id=(c,),
                    device_id_type=pl.DeviceIdType.MESH,
                )
                rdma.start()
            pl.when(my != c)(p2_send)

        for j in range(N_DEV):
            def p2_take(j=j):
                recv = pltpu.make_async_remote_copy(
                    src_ref=bcast_buf.at[j],
                    dst_ref=bcast_buf.at[j],
                    send_sem=bsend_sems.at[j],
                    recv_sem=brecv_sems.at[j],
                    device_id=(j,),
                    device_id_type=pl.DeviceIdType.MESH,
                )
                recv.wait_recv()
                acc[CH * j:CH * (j + 1), :] = (
                    bcast_buf[j].astype(jnp.float32))
            pl.when(my != j)(p2_take)

        for c in range(N_DEV):
            def drain(c=c):
                s1 = pltpu.make_async_remote_copy(
                    src_ref=pbuf.at[pl.ds(CH * c, CH), :],
                    dst_ref=scatter_buf.at[my],
                    send_sem=ssend_sems.at[c],
                    recv_sem=srecv_sems.at[my],
                    device_id=(c,),
                    device_id_type=pl.DeviceIdType.MESH,
                )
                s1.wait_send()
                s2 = pltpu.make_async_remote_copy(
                    src_ref=bbuf,
                    dst_ref=bcast_buf.at[my],
                    send_sem=bsend_sems.at[c],
                    recv_sem=brecv_sems.at[my],
                    device_id=(c,),
                    device_id_type=pl.DeviceIdType.MESH,
                )
                s2.wait_send()
            pl.when(my != c)(drain)

    return pl.pallas_call(
        body,
        out_shape=jax.ShapeDtypeStruct((1, SQ, D), jnp.float32),
        in_specs=[
            pl.BlockSpec(memory_space=pltpu.VMEM),
            pl.BlockSpec(memory_space=pltpu.VMEM),
            pl.BlockSpec(memory_space=pltpu.VMEM),
            pl.BlockSpec(memory_space=pl.ANY),
            pl.BlockSpec(memory_space=pl.ANY),
        ],
        out_specs=pl.BlockSpec(memory_space=pltpu.VMEM),
        scratch_shapes=[
            pltpu.VMEM((2, D, KV_COLS), jnp.float32),
            pltpu.VMEM((SQ, D), jnp.float32),
            pltpu.VMEM((SQ, D), jnp.bfloat16),
            pltpu.VMEM((N_DEV, CH, D), jnp.bfloat16),
            pltpu.VMEM((CH, D), jnp.bfloat16),
            pltpu.VMEM((N_DEV, CH, D), jnp.bfloat16),
            pltpu.SemaphoreType.DMA((3,)),
            pltpu.SemaphoreType.DMA((N_DEV,)),
            pltpu.SemaphoreType.DMA((N_DEV,)),
            pltpu.SemaphoreType.DMA((N_DEV,)),
            pltpu.SemaphoreType.DMA((N_DEV,)),
        ],
        compiler_params=pltpu.CompilerParams(collective_id=0),
    )(x, Wq, Wo, Wk, Wv)


# device time: 25520 ns/iter; 1.0630x vs baseline; 1.0608x over previous
import jax
import jax.numpy as jnp
from jax import lax
from jax.experimental import pallas as pl
from jax.experimental.pallas import tpu as pltpu

N_DEV = 8
SQ = 256
D = 1024
DH = 128
HQ_PER = 8
KV_COLS = 256
CH = SQ // N_DEV
SCALE = 0.08838834764831843


def kernel(x, Wq, Wo, Wk, Wv):
    i = lax.axis_index("i")
    Wk_s = lax.dynamic_slice(Wk, (0, i * KV_COLS), (D, KV_COLS))
    Wv_s = lax.dynamic_slice(Wv, (0, i * KV_COLS), (D, KV_COLS))

    def body(x_ref, wq_ref, wo_ref, wk_ref, wv_ref, out_ref,
             pbuf32, pbuf, scatter_buf, bbuf, bcast_buf,
             ssend_sems, srecv_sems, bsend_sems, brecv_sems):
        my = lax.axis_index("i")
        acc = out_ref.at[0]
        my_rows = pl.ds(CH * my, CH)

        barrier = pltpu.get_barrier_semaphore()
        for p in range(N_DEV):
            pl.when(my != p)(lambda p=p: pl.semaphore_signal(
                barrier, inc=1,
                device_id=(p,), device_id_type=pl.DeviceIdType.MESH,
            ))

        xv = x_ref[0, :, :].astype(jnp.bfloat16)
        q = jnp.dot(xv, wq_ref[...].astype(jnp.bfloat16),
                    preferred_element_type=jnp.float32)
        k = jnp.dot(xv, wk_ref[...].astype(jnp.bfloat16),
                    preferred_element_type=jnp.float32)
        v = jnp.dot(xv, wv_ref[...].astype(jnp.bfloat16),
                    preferred_element_type=jnp.float32)

        outs = []
        for h in range(HQ_PER):
            qh = q[:, h * DH:(h + 1) * DH].astype(jnp.bfloat16)
            g = h // 4
            kh = k[:, g * DH:(g + 1) * DH].astype(jnp.bfloat16)
            vh = v[:, g * DH:(g + 1) * DH].astype(jnp.bfloat16)
            s = lax.dot_general(
                qh, kh, (((1,), (1,)), ((), ())),
                preferred_element_type=jnp.float32,
            ) * SCALE
            m = jnp.max(s, axis=-1, keepdims=True)
            p = jnp.exp(s - m)
            l = jnp.sum(p, axis=-1, keepdims=True)
            ph = p.astype(jnp.bfloat16)
            outs.append(jnp.dot(ph, vh, preferred_element_type=jnp.float32) / l)
        o = jnp.concatenate(outs, axis=1).astype(jnp.bfloat16)
        wo_bf = wo_ref[...].astype(jnp.bfloat16)

        pl.semaphore_wait(barrier, N_DEV - 1)

        for c in range(N_DEV):
            rows = slice(CH * c, CH * (c + 1))
            chunk = jnp.dot(o[rows, :], wo_bf,
                            preferred_element_type=jnp.float32)
            pbuf32[rows, :] = chunk
            pbuf[rows, :] = chunk.astype(jnp.bfloat16)

            def p1_send(c=c):
                rdma = pltpu.make_async_remote_copy(
                    src_ref=pbuf.at[pl.ds(CH * c, CH), :],
                    dst_ref=scatter_buf.at[my],
                    send_sem=ssend_sems.at[c],
                    recv_sem=srecv_sems.at[my],
                    device_id=(c,),
                    device_id_type=pl.DeviceIdType.MESH,
                )
                rdma.start()
            pl.when(my != c)(p1_send)

        scatter_buf[my] = jnp.zeros((CH, D), jnp.bfloat16)
        for j in range(N_DEV):
            def p1_wait(j=j):
                recv = pltpu.make_async_remote_copy(
                    src_ref=scatter_buf.at[j],
                    dst_ref=scatter_buf.at[j],
                    send_sem=ssend_sems.at[j],
                    recv_sem=srecv_sems.at[j],
                    device_id=(j,),
                    device_id_type=pl.DeviceIdType.MESH,
                )
                recv.wait_recv()
            pl.when(my != j)(p1_wait)

        red = pbuf32[my_rows, :]
        for j in range(N_DEV):
            red = red + scatter_buf[j].astype(jnp.float32)
        acc[my_rows, :] = red
        bbuf[...] = red.astype(jnp.bfloat16)

        for c in range(N_DEV):
            def p2_send(c=c):
                rdma = pltpu.make_async_remote_copy(
                    src_ref=bbuf,
                    dst_ref=bcast_buf.at[my],
                    send_sem=bsend_sems.at[c],
                    recv_sem=brecv_sems.at[my],
                    device_id=(c,),
                    device_id_type=pl.DeviceIdType.MESH,
                )
                rdma.start()
            pl.when(my != c)(p2_send)

        for j in range(N_DEV):
            def p2_take(j=j):
                recv = pltpu.make_async_remote_copy(
                    src_ref=bcast_buf.at[j],
                    dst_ref=bcast_buf.at[j],
                    send_sem=bsend_sems.at[j],
                    recv_sem=brecv_sems.at[j],
                    device_id=(j,),
                    device_id_type=pl.DeviceIdType.MESH,
                )
                recv.wait_recv()
                acc[CH * j:CH * (j + 1), :] = (
                    bcast_buf[j].astype(jnp.float32))
            pl.when(my != j)(p2_take)

        for c in range(N_DEV):
            def drain(c=c):
                s1 = pltpu.make_async_remote_copy(
                    src_ref=pbuf.at[pl.ds(CH * c, CH), :],
                    dst_ref=scatter_buf.at[my],
                    send_sem=ssend_sems.at[c],
                    recv_sem=srecv_sems.at[my],
                    device_id=(c,),
                    device_id_type=pl.DeviceIdType.MESH,
                )
                s1.wait_send()
                s2 = pltpu.make_async_remote_copy(
                    src_ref=bbuf,
                    dst_ref=bcast_buf.at[my],
                    send_sem=bsend_sems.at[c],
                    recv_sem=brecv_sems.at[my],
                    device_id=(c,),
                    device_id_type=pl.DeviceIdType.MESH,
                )
                s2.wait_send()
            pl.when(my != c)(drain)

    return pl.pallas_call(
        body,
        out_shape=jax.ShapeDtypeStruct((1, SQ, D), jnp.float32),
        in_specs=[pl.BlockSpec(memory_space=pltpu.VMEM)] * 5,
        out_specs=pl.BlockSpec(memory_space=pltpu.VMEM),
        scratch_shapes=[
            pltpu.VMEM((SQ, D), jnp.float32),
            pltpu.VMEM((SQ, D), jnp.bfloat16),
            pltpu.VMEM((N_DEV, CH, D), jnp.bfloat16),
            pltpu.VMEM((CH, D), jnp.bfloat16),
            pltpu.VMEM((N_DEV, CH, D), jnp.bfloat16),
            pltpu.SemaphoreType.DMA((N_DEV,)),
            pltpu.SemaphoreType.DMA((N_DEV,)),
            pltpu.SemaphoreType.DMA((N_DEV,)),
            pltpu.SemaphoreType.DMA((N_DEV,)),
        ],
        compiler_params=pltpu.CompilerParams(collective_id=0),
    )(x, Wq, Wo, Wk_s, Wv_s)
